# baseline (speedup 1.0000x reference)
import jax
import jax.numpy as jnp
from jax import lax
from jax.experimental import pallas as pl
from jax.experimental.pallas import tpu as pltpu
from jax.experimental.pallas import tpu_sc as plsc

B = 32
S = 256
T = 1024
MASK = 768
D = 768
GCHUNK = 64
MCHUNK = 16
LANES = 16
NC = 2
NS = 16


def _body(inputs_hbm, mst_hbm, un_src, un_dst, mk_dst, meta_un, meta_mk, out,
          un_src_v, un_dst_v, mk_dst_v, meta_un_v, meta_mk_v,
          mst_buf, buf0, buf1,
          gsem0, gsem1, wsem0, wsem1, msem):
    wid = lax.axis_index("s") * NC + lax.axis_index("c")
    inp_w = inputs_hbm.at[pl.ds(wid * S, S)]
    out_w = out.at[pl.ds(wid * T, T)]

    pltpu.sync_copy(un_src, un_src_v)
    pltpu.sync_copy(un_dst, un_dst_v)
    pltpu.sync_copy(mk_dst, mk_dst_v)
    pltpu.sync_copy(meta_un, meta_un_v)
    pltpu.sync_copy(meta_mk, meta_mk_v)

    pltpu.sync_copy(mst_hbm, mst_buf)

    n_un = meta_un_v[pl.ds(0, LANES)][0]
    n_mk = meta_mk_v[pl.ds(0, LANES)][0]

    def mk_fire(c, carry):
        pltpu.make_async_copy(mst_buf, out_w.at[mk_dst_v.at[c]], msem).start()
        return carry

    lax.fori_loop(0, n_mk, mk_fire, 0)

    def un_pair(p, carry):
        c0 = 2 * p
        c1 = c0 + 1

        def g(c, buf, sem):
            return pltpu.make_async_copy(inp_w.at[un_src_v.at[c]], buf, sem)

        def w(c, buf, sem):
            return pltpu.make_async_copy(buf, out_w.at[un_dst_v.at[c]], sem)

        g(c0, buf0, gsem0).start()
        pl.when(c1 < n_un)(lambda: g(c1, buf1, gsem1).start())
        g(c0, buf0, gsem0).wait()
        w(c0, buf0, wsem0).start()

        def mid():
            g(c1, buf1, gsem1).wait()
            w(c1, buf1, wsem1).start()

        pl.when(c1 < n_un)(mid)
        w(c0, buf0, wsem0).wait()
        pl.when(c1 < n_un)(lambda: w(c1, buf1, wsem1).wait())
        return carry

    lax.fori_loop(0, (n_un + 1) // 2, un_pair, 0)

    def mk_drain(c, carry):
        pltpu.make_async_copy(mst_buf, out_w.at[mk_dst_v.at[0]], msem).wait()
        return carry

    lax.fori_loop(0, n_mk, mk_drain, 0)


def kernel(inputs, mask_indices, un_masked_indices, mst):
    idx = jnp.concatenate([mask_indices, un_masked_indices]).astype(jnp.int32)
    pos = jnp.arange(T, dtype=jnp.int32)
    is_un = idx >= MASK
    cnt_un = jnp.sum(is_un.astype(jnp.int32))
    cnt_mk = T - cnt_un

    ord_un = jnp.argsort(jnp.logical_not(is_un), stable=True).astype(jnp.int32)
    ksel = jnp.where(pos < cnt_un, pos, pos % jnp.maximum(cnt_un, 1))
    un_dst = ord_un[ksel]
    un_src = (idx[ord_un] - MASK)[ksel]

    ord_mk = jnp.argsort(is_un, stable=True).astype(jnp.int32)
    ksel2 = jnp.where(pos < cnt_mk, pos, pos % jnp.maximum(cnt_mk, 1))
    mk_dst = ord_mk[ksel2]

    meta_un = jnp.zeros((LANES,), jnp.int32).at[0].set(
        (cnt_un + GCHUNK - 1) // GCHUNK)
    meta_mk = jnp.zeros((LANES,), jnp.int32).at[0].set(
        (cnt_mk + MCHUNK - 1) // MCHUNK)

    mesh = plsc.VectorSubcoreMesh(core_axis_name="c", subcore_axis_name="s")
    out = pl.kernel(
        _body,
        mesh=mesh,
        out_type=jax.ShapeDtypeStruct((B * T, D), inputs.dtype),
        scratch_types=[
            pltpu.VMEM((T // GCHUNK, GCHUNK), jnp.int32),
            pltpu.VMEM((T // GCHUNK, GCHUNK), jnp.int32),
            pltpu.VMEM((T // MCHUNK, MCHUNK), jnp.int32),
            pltpu.VMEM((LANES,), jnp.int32),
            pltpu.VMEM((LANES,), jnp.int32),
            pltpu.VMEM((MCHUNK, D), jnp.float32),
            pltpu.VMEM((GCHUNK, D), jnp.float32),
            pltpu.VMEM((GCHUNK, D), jnp.float32),
            pltpu.SemaphoreType.DMA,
            pltpu.SemaphoreType.DMA,
            pltpu.SemaphoreType.DMA,
            pltpu.SemaphoreType.DMA,
            pltpu.SemaphoreType.DMA,
        ],
    )(inputs.reshape(B * S, D),
      jnp.broadcast_to(mst.reshape(1, D).astype(inputs.dtype), (MCHUNK, D)),
      un_src.reshape(T // GCHUNK, GCHUNK),
      un_dst.reshape(T // GCHUNK, GCHUNK),
      mk_dst.reshape(T // MCHUNK, MCHUNK),
      meta_un, meta_mk)
    return out.reshape(B, T, D)
